# baseline (speedup 1.0000x reference)
import functools

import jax
import jax.numpy as jnp
from jax import lax
from jax.experimental import pallas as pl
from jax.experimental.pallas import tpu as pltpu
from jax.experimental.pallas import tpu_sc as plsc

_VOCAB = 100000
_DIM = 64
_K = 4
_N = 16384

_NC = 2
_NS = 16
_NW = _NC * _NS
_BPW = _N // _NW
_XCH = 4
_XB = _BPW // _XCH
_LANE = 16

_MESH = plsc.VectorSubcoreMesh(core_axis_name="c", subcore_axis_name="s")
_PARAMS = pltpu.CompilerParams(use_tc_tiling_on_sc=False,
                               needs_layout_passes=False)


def _wid_base():
    wid = lax.axis_index("s") * _NC + lax.axis_index("c")
    return wid * _BPW


@functools.partial(
    pl.kernel,
    mesh=_MESH,
    compiler_params=_PARAMS,
    out_type=jax.ShapeDtypeStruct((_N, _DIM), jnp.float32),
    scratch_types=[
        pltpu.VMEM((_BPW,), jnp.int32),
        pltpu.VMEM((2, _BPW // 2, _DIM), jnp.float32),
        pltpu.SemaphoreType.DMA,
        pltpu.SemaphoreType.DMA,
    ],
)
def _gather_funcs(ranks_hbm, fw_hbm, out_hbm, idx_v, f_v, sem0, sem1):
    base = _wid_base()
    pltpu.sync_copy(ranks_hbm.at[pl.ds(base, _BPW)], idx_v)
    half = _BPW // 2
    sems = (sem0, sem1)
    copies = [None, None]
    copies[0] = pltpu.async_copy(
        fw_hbm.at[idx_v.at[pl.ds(0, half)]], f_v.at[0], sems[0])
    for i in range(2):
        if i + 1 < 2:
            copies[i + 1] = pltpu.async_copy(
                fw_hbm.at[idx_v.at[pl.ds((i + 1) * half, half)]],
                f_v.at[i + 1], sems[i + 1])
        copies[i].wait()
        pltpu.sync_copy(f_v.at[i], out_hbm.at[pl.ds(base + i * half, half)])


@functools.partial(
    pl.kernel,
    mesh=_MESH,
    compiler_params=_PARAMS,
    out_type=jax.ShapeDtypeStruct((_N, _K, _DIM), jnp.float32),
    scratch_types=[
        pltpu.VMEM((_BPW,), jnp.int32),
        pltpu.VMEM((2, _XB, _K, _DIM), jnp.float32),
        pltpu.SemaphoreType.DMA,
        pltpu.SemaphoreType.DMA,
    ],
)
def _gather_x(ranks_hbm, mx_hbm, out_hbm, idx_v, x_v, sem0, sem1):
    base = _wid_base()
    pltpu.sync_copy(ranks_hbm.at[pl.ds(base, _BPW)], idx_v)
    sems = (sem0, sem1)
    copies = [None, None]
    copies[0] = pltpu.async_copy(
        mx_hbm.at[idx_v.at[pl.ds(0, _XB)]], x_v.at[0], sems[0])
    for i in range(_XCH):
        if i + 1 < _XCH:
            copies[(i + 1) % 2] = pltpu.async_copy(
                mx_hbm.at[idx_v.at[pl.ds((i + 1) * _XB, _XB)]],
                x_v.at[(i + 1) % 2], sems[(i + 1) % 2])
        copies[i % 2].wait()
        pltpu.sync_copy(x_v.at[i % 2],
                        out_hbm.at[pl.ds(base + i * _XB, _XB)])


@functools.partial(
    pl.kernel,
    mesh=_MESH,
    compiler_params=_PARAMS,
    out_type=jax.ShapeDtypeStruct((_N, _K), jnp.float32),
    scratch_types=[
        pltpu.VMEM((_BPW,), jnp.int32),
        pltpu.VMEM((_BPW,), jnp.int32),
        pltpu.VMEM((_BPW, 16), jnp.float32),
        pltpu.VMEM((_BPW, _K), jnp.float32),
        pltpu.SemaphoreType.DMA,
    ],
)
def _gather_m(ranks_hbm, mm_hbm, out_hbm, idx_v, idxq_v, g_v, m_v, sem):
    base = _wid_base()
    pltpu.sync_copy(ranks_hbm.at[pl.ds(base, _BPW)], idx_v)
    for i in range(_BPW // _LANE):
        r = idx_v[pl.ds(i * _LANE, _LANE)]
        idxq_v[pl.ds(i * _LANE, _LANE)] = lax.shift_right_logical(r, 2)
    cm = pltpu.async_copy(mm_hbm.at[idxq_v], g_v, sem)
    cm.wait()
    lanes = lax.iota(jnp.int32, _LANE)
    for i in range(_BPW // _LANE):
        jvec = lanes + i * _LANE
        r = idx_v[pl.ds(i * _LANE, _LANE)]
        off = lax.shift_left(lax.bitwise_and(r, 3), 2)
        for t in range(_K):
            vals = plsc.load_gather(g_v, [jvec, off + t])
            plsc.store_scatter(m_v, [jvec, lanes * 0 + t], vals)
    pltpu.sync_copy(m_v, out_hbm.at[pl.ds(base, _BPW)])


@jax.jit
def _fire_word(ranks, func_weight, measure_x, measure_m):
    mm16 = measure_m.reshape(_VOCAB // 4, 16)
    funcs = _gather_funcs(ranks, func_weight)
    meas_x = _gather_x(ranks, measure_x)
    meas_m = _gather_m(ranks, mm16)
    return funcs, meas_x, meas_m


def kernel(ranks, func_weight, measure_x, measure_m):
    return _fire_word(ranks, func_weight, measure_x, measure_m)


# speedup vs baseline: 1.4902x; 1.4902x over previous
import functools

import jax
import jax.numpy as jnp
from jax import lax
from jax.experimental import pallas as pl
from jax.experimental.pallas import tpu as pltpu
from jax.experimental.pallas import tpu_sc as plsc

_VOCAB = 100000
_DIM = 64
_K = 4
_N = 16384

_NC = 2
_NS = 16
_NW = _NC * _NS
_BPW = _N // _NW
_LANE = 16

_XCH = 8
_XB = _BPW // _XCH
_FCH = 4
_FB = _BPW // _FCH
_MCH = 4
_MB = _BPW // _MCH

_MESH = plsc.VectorSubcoreMesh(core_axis_name="c", subcore_axis_name="s")
_PARAMS = pltpu.CompilerParams(use_tc_tiling_on_sc=True,
                               needs_layout_passes=False)


@functools.partial(
    pl.kernel,
    mesh=_MESH,
    compiler_params=_PARAMS,
    out_type=(
        jax.ShapeDtypeStruct((_DIM, _N), jnp.float32),
        jax.ShapeDtypeStruct((_N, _K * _DIM), jnp.float32),
        jax.ShapeDtypeStruct((_K, _N), jnp.float32),
    ),
    scratch_types=[
        pltpu.VMEM((_BPW,), jnp.int32),
        pltpu.VMEM((_BPW,), jnp.int32),
        pltpu.VMEM((_BPW,), jnp.int32),
        pltpu.VMEM((2, _XB, _K * _DIM), jnp.float32),
        pltpu.VMEM((2, _FB, 128), jnp.float32),
        pltpu.VMEM((2, _MB, 128), jnp.float32),
        pltpu.VMEM((_DIM, _FB), jnp.float32),
        pltpu.VMEM((_K, _MB), jnp.float32),
        pltpu.SemaphoreType.DMA,
        pltpu.SemaphoreType.DMA,
        pltpu.SemaphoreType.DMA,
        pltpu.SemaphoreType.DMA,
        pltpu.SemaphoreType.DMA,
        pltpu.SemaphoreType.DMA,
    ],
)
def _fire_word_sc(ranks_hbm, fw_hbm, mx_hbm, mm_hbm,
                  out_f_hbm, out_x_hbm, out_m_hbm,
                  idx_v, idxf_v, idxm_v, x_v, g_v, h_v, ft_v, mt_v,
                  sx0, sx1, sf0, sf1, sm0, sm1):
    wid = lax.axis_index("s") * _NC + lax.axis_index("c")
    base = wid * _BPW
    pltpu.sync_copy(ranks_hbm.at[pl.ds(base, _BPW)], idx_v)

    def _shift(i, _):
        r = idx_v[pl.ds(i * _LANE, _LANE)]
        idxf_v[pl.ds(i * _LANE, _LANE)] = lax.shift_right_logical(r, 1)
        idxm_v[pl.ds(i * _LANE, _LANE)] = lax.shift_right_logical(r, 5)
        return ()

    lax.fori_loop(0, _BPW // _LANE, _shift, (), unroll=False)

    xsems = (sx0, sx1)
    fsems = (sf0, sf1)
    msems = (sm0, sm1)
    xcp = [None, None]
    fcp = [None, None]
    mcp = [None, None]
    xcp[0] = pltpu.async_copy(
        mx_hbm.at[idx_v.at[pl.ds(0, _XB)]], x_v.at[0], xsems[0])
    fcp[0] = pltpu.async_copy(
        fw_hbm.at[idxf_v.at[pl.ds(0, _FB)]], g_v.at[0], fsems[0])
    mcp[0] = pltpu.async_copy(
        mm_hbm.at[idxm_v.at[pl.ds(0, _MB)]], h_v.at[0], msems[0])

    lanes = lax.iota(jnp.int32, _LANE)

    for i in range(_XCH):
        if i + 1 < _XCH:
            xcp[(i + 1) % 2] = pltpu.async_copy(
                mx_hbm.at[idx_v.at[pl.ds((i + 1) * _XB, _XB)]],
                x_v.at[(i + 1) % 2], xsems[(i + 1) % 2])
        xcp[i % 2].wait()
        pltpu.sync_copy(x_v.at[i % 2],
                        out_x_hbm.at[pl.ds(base + i * _XB, _XB)])

    for i in range(_FCH):
        if i + 1 < _FCH:
            fcp[(i + 1) % 2] = pltpu.async_copy(
                fw_hbm.at[idxf_v.at[pl.ds((i + 1) * _FB, _FB)]],
                g_v.at[(i + 1) % 2], fsems[(i + 1) % 2])
        fcp[i % 2].wait()
        gbuf = g_v.at[i % 2]

        def _fext(k, _):
            rg = k // _DIM
            d = k % _DIM
            rows = lanes + rg * _LANE
            r = idx_v[pl.ds(i * _FB + rg * _LANE, _LANE)]
            off = lax.shift_left(lax.bitwise_and(r, 1), 6) + d
            vals = plsc.load_gather(gbuf, [rows, off])
            plsc.store_scatter(ft_v, [lanes * 0 + d, rows], vals)
            return ()

        lax.fori_loop(0, (_FB // _LANE) * _DIM, _fext, (), unroll=False)
        pltpu.sync_copy(ft_v, out_f_hbm.at[:, pl.ds(base + i * _FB, _FB)])

    for i in range(_MCH):
        if i + 1 < _MCH:
            mcp[(i + 1) % 2] = pltpu.async_copy(
                mm_hbm.at[idxm_v.at[pl.ds((i + 1) * _MB, _MB)]],
                h_v.at[(i + 1) % 2], msems[(i + 1) % 2])
        mcp[i % 2].wait()
        hbuf = h_v.at[i % 2]

        def _mext(k, _):
            rg = k // _K
            t = k % _K
            rows = lanes + rg * _LANE
            r = idx_v[pl.ds(i * _MB + rg * _LANE, _LANE)]
            off = lax.shift_left(lax.bitwise_and(r, 31), 2) + t
            vals = plsc.load_gather(hbuf, [rows, off])
            plsc.store_scatter(mt_v, [lanes * 0 + t, rows], vals)
            return ()

        lax.fori_loop(0, (_MB // _LANE) * _K, _mext, (), unroll=False)
        pltpu.sync_copy(mt_v, out_m_hbm.at[:, pl.ds(base + i * _MB, _MB)])


@jax.jit
def _fire_word(ranks, func_weight, measure_x, measure_m):
    fw2 = func_weight.reshape(_VOCAB // 2, 128)
    mx2 = measure_x.reshape(_VOCAB, _K * _DIM)
    mm2 = measure_m.reshape(_VOCAB * _K // 128, 128)
    f_t, x_rows, m_t = _fire_word_sc(ranks, fw2, mx2, mm2)
    return (f_t.T, x_rows.reshape(_N, _K, _DIM), m_t.T)


def kernel(ranks, func_weight, measure_x, measure_m):
    return _fire_word(ranks, func_weight, measure_x, measure_m)


# speedup vs baseline: 1.6096x; 1.0801x over previous
import functools

import jax
import jax.numpy as jnp
from jax import lax
from jax.experimental import pallas as pl
from jax.experimental.pallas import tpu as pltpu
from jax.experimental.pallas import tpu_sc as plsc

_VOCAB = 100000
_DIM = 64
_K = 4
_N = 16384

_NC = 2
_NS = 16
_NW = _NC * _NS
_BPW = _N // _NW
_LANE = 16

_XCH = 8
_XB = _BPW // _XCH
_FCH = 4
_FB = _BPW // _FCH
_MCH = 4
_MB = _BPW // _MCH

_MESH = plsc.VectorSubcoreMesh(core_axis_name="c", subcore_axis_name="s")
_PARAMS = pltpu.CompilerParams(use_tc_tiling_on_sc=True,
                               needs_layout_passes=False)


def _base():
    wid = lax.axis_index("s") * _NC + lax.axis_index("c")
    return wid * _BPW


@functools.partial(
    pl.kernel,
    mesh=_MESH,
    compiler_params=_PARAMS,
    out_type=jax.ShapeDtypeStruct((_N, _K * _DIM), jnp.float32),
    scratch_types=[
        pltpu.VMEM((_BPW,), jnp.int32),
        pltpu.VMEM((2, _XB, _K * _DIM), jnp.float32),
        pltpu.SemaphoreType.DMA,
        pltpu.SemaphoreType.DMA,
    ],
)
def _gather_x(ranks_hbm, mx_hbm, out_hbm, idx_v, x_v, s0, s1):
    base = _base()
    pltpu.sync_copy(ranks_hbm.at[pl.ds(base, _BPW)], idx_v)
    sems = (s0, s1)
    cp = [None, None]
    cp[0] = pltpu.async_copy(
        mx_hbm.at[idx_v.at[pl.ds(0, _XB)]], x_v.at[0], sems[0])
    for i in range(_XCH):
        if i + 1 < _XCH:
            cp[(i + 1) % 2] = pltpu.async_copy(
                mx_hbm.at[idx_v.at[pl.ds((i + 1) * _XB, _XB)]],
                x_v.at[(i + 1) % 2], sems[(i + 1) % 2])
        cp[i % 2].wait()
        pltpu.sync_copy(x_v.at[i % 2],
                        out_hbm.at[pl.ds(base + i * _XB, _XB)])


@functools.partial(
    pl.kernel,
    mesh=_MESH,
    compiler_params=_PARAMS,
    out_type=jax.ShapeDtypeStruct((_DIM, _N), jnp.float32),
    scratch_types=[
        pltpu.VMEM((_BPW,), jnp.int32),
        pltpu.VMEM((_BPW,), jnp.int32),
        pltpu.VMEM((2, _FB, 128), jnp.float32),
        pltpu.VMEM((_DIM, _FB), jnp.float32),
        pltpu.SemaphoreType.DMA,
        pltpu.SemaphoreType.DMA,
    ],
)
def _gather_f(ranks_hbm, fw_hbm, out_hbm, idx_v, idxf_v, g_v, ft_v, s0, s1):
    base = _base()
    pltpu.sync_copy(ranks_hbm.at[pl.ds(base, _BPW)], idx_v)

    def _shift(i, _):
        r = idx_v[pl.ds(i * _LANE, _LANE)]
        idxf_v[pl.ds(i * _LANE, _LANE)] = lax.shift_right_logical(r, 1)
        return ()

    lax.fori_loop(0, _BPW // _LANE, _shift, (), unroll=False)
    sems = (s0, s1)
    cp = [None, None]
    cp[0] = pltpu.async_copy(
        fw_hbm.at[idxf_v.at[pl.ds(0, _FB)]], g_v.at[0], sems[0])
    lanes = lax.iota(jnp.int32, _LANE)
    for i in range(_FCH):
        if i + 1 < _FCH:
            cp[(i + 1) % 2] = pltpu.async_copy(
                fw_hbm.at[idxf_v.at[pl.ds((i + 1) * _FB, _FB)]],
                g_v.at[(i + 1) % 2], sems[(i + 1) % 2])
        cp[i % 2].wait()
        gbuf = g_v.at[i % 2]

        def _fext(k, _):
            rg = k // _DIM
            d = k % _DIM
            rows = lanes + rg * _LANE
            r = idx_v[pl.ds(i * _FB + rg * _LANE, _LANE)]
            off = lax.shift_left(lax.bitwise_and(r, 1), 6) + d
            vals = plsc.load_gather(gbuf, [rows, off])
            plsc.store_scatter(ft_v, [lanes * 0 + d, rows], vals)
            return ()

        lax.fori_loop(0, (_FB // _LANE) * _DIM, _fext, (), unroll=False)
        pltpu.sync_copy(ft_v, out_hbm.at[:, pl.ds(base + i * _FB, _FB)])


@functools.partial(
    pl.kernel,
    mesh=_MESH,
    compiler_params=_PARAMS,
    out_type=jax.ShapeDtypeStruct((_K, _N), jnp.float32),
    scratch_types=[
        pltpu.VMEM((_BPW,), jnp.int32),
        pltpu.VMEM((_BPW,), jnp.int32),
        pltpu.VMEM((2, _MB, 128), jnp.float32),
        pltpu.VMEM((_K, _MB), jnp.float32),
        pltpu.SemaphoreType.DMA,
        pltpu.SemaphoreType.DMA,
    ],
)
def _gather_m(ranks_hbm, mm_hbm, out_hbm, idx_v, idxm_v, h_v, mt_v, s0, s1):
    base = _base()
    pltpu.sync_copy(ranks_hbm.at[pl.ds(base, _BPW)], idx_v)

    def _shift(i, _):
        r = idx_v[pl.ds(i * _LANE, _LANE)]
        idxm_v[pl.ds(i * _LANE, _LANE)] = lax.shift_right_logical(r, 5)
        return ()

    lax.fori_loop(0, _BPW // _LANE, _shift, (), unroll=False)
    sems = (s0, s1)
    cp = [None, None]
    cp[0] = pltpu.async_copy(
        mm_hbm.at[idxm_v.at[pl.ds(0, _MB)]], h_v.at[0], sems[0])
    lanes = lax.iota(jnp.int32, _LANE)
    for i in range(_MCH):
        if i + 1 < _MCH:
            cp[(i + 1) % 2] = pltpu.async_copy(
                mm_hbm.at[idxm_v.at[pl.ds((i + 1) * _MB, _MB)]],
                h_v.at[(i + 1) % 2], sems[(i + 1) % 2])
        cp[i % 2].wait()
        hbuf = h_v.at[i % 2]

        def _mext(k, _):
            rg = k // _K
            t = k % _K
            rows = lanes + rg * _LANE
            r = idx_v[pl.ds(i * _MB + rg * _LANE, _LANE)]
            off = lax.shift_left(lax.bitwise_and(r, 31), 2) + t
            vals = plsc.load_gather(hbuf, [rows, off])
            plsc.store_scatter(mt_v, [lanes * 0 + t, rows], vals)
            return ()

        lax.fori_loop(0, (_MB // _LANE) * _K, _mext, (), unroll=False)
        pltpu.sync_copy(mt_v, out_hbm.at[:, pl.ds(base + i * _MB, _MB)])


@jax.jit
def _fire_word(ranks, func_weight, measure_x, measure_m):
    fw2 = func_weight.reshape(_VOCAB // 2, 128)
    mx2 = measure_x.reshape(_VOCAB, _K * _DIM)
    mm_flat = lax.optimization_barrier(measure_m.reshape(_VOCAB * _K))
    mm2 = mm_flat.reshape(_VOCAB * _K // 128, 128)
    x_rows = _gather_x(ranks, mx2)
    f_t = _gather_f(ranks, fw2)
    m_t = _gather_m(ranks, mm2)
    return (f_t.T, x_rows.reshape(_N, _K, _DIM), m_t.T)


def kernel(ranks, func_weight, measure_x, measure_m):
    return _fire_word(ranks, func_weight, measure_x, measure_m)


# speedup vs baseline: 1.6902x; 1.0501x over previous
import functools

import jax
import jax.numpy as jnp
from jax import lax
from jax.experimental import pallas as pl
from jax.experimental.pallas import tpu as pltpu
from jax.experimental.pallas import tpu_sc as plsc

_VOCAB = 100000
_DIM = 64
_K = 4
_N = 16384

_NC = 2
_NS = 16
_NW = _NC * _NS
_BPW = _N // _NW
_LANE = 16

_XCH = 8
_XB = _BPW // _XCH
_FCH = 4
_FB = _BPW // _FCH
_MCH = 4
_MB = _BPW // _MCH

_MESH = plsc.VectorSubcoreMesh(core_axis_name="c", subcore_axis_name="s")
_PARAMS = pltpu.CompilerParams(use_tc_tiling_on_sc=True,
                               needs_layout_passes=False)


def _base():
    wid = lax.axis_index("s") * _NC + lax.axis_index("c")
    return wid * _BPW


@functools.partial(
    pl.kernel,
    mesh=_MESH,
    compiler_params=_PARAMS,
    out_type=jax.ShapeDtypeStruct((_N, _K * _DIM), jnp.float32),
    scratch_types=[
        pltpu.VMEM((_BPW,), jnp.int32),
        pltpu.VMEM((3, _XB, _K * _DIM), jnp.float32),
        pltpu.SemaphoreType.DMA,
        pltpu.SemaphoreType.DMA,
        pltpu.SemaphoreType.DMA,
        pltpu.SemaphoreType.DMA,
    ],
)
def _gather_x(ranks_hbm, mx_hbm, out_hbm, idx_v, x_v, s0, s1, s2, so):
    base = _base()
    pltpu.sync_copy(ranks_hbm.at[pl.ds(base, _BPW)], idx_v)
    sems = (s0, s1, s2)
    cp = [None] * _XCH
    ocp = [None] * _XCH
    for i in range(2):
        cp[i] = pltpu.async_copy(
            mx_hbm.at[idx_v.at[pl.ds(i * _XB, _XB)]], x_v.at[i], sems[i])
    for i in range(_XCH):
        cp[i].wait()
        if i + 2 < _XCH:
            if i >= 1:
                ocp[i - 1].wait()
            cp[i + 2] = pltpu.async_copy(
                mx_hbm.at[idx_v.at[pl.ds((i + 2) * _XB, _XB)]],
                x_v.at[(i + 2) % 3], sems[(i + 2) % 3])
        ocp[i] = pltpu.async_copy(
            x_v.at[i % 3], out_hbm.at[pl.ds(base + i * _XB, _XB)], so)
    for i in range(max(_XCH - 3, 0), _XCH):
        ocp[i].wait()


@functools.partial(
    pl.kernel,
    mesh=_MESH,
    compiler_params=_PARAMS,
    out_type=jax.ShapeDtypeStruct((_DIM, _N), jnp.float32),
    scratch_types=[
        pltpu.VMEM((_BPW,), jnp.int32),
        pltpu.VMEM((_BPW,), jnp.int32),
        pltpu.VMEM((2, _FB, 128), jnp.float32),
        pltpu.VMEM((_DIM, _FB), jnp.float32),
        pltpu.SemaphoreType.DMA,
        pltpu.SemaphoreType.DMA,
    ],
)
def _gather_f(ranks_hbm, fw_hbm, out_hbm, idx_v, idxf_v, g_v, ft_v, s0, s1):
    base = _base()
    pltpu.sync_copy(ranks_hbm.at[pl.ds(base, _BPW)], idx_v)

    def _shift(i, _):
        r = idx_v[pl.ds(i * _LANE, _LANE)]
        idxf_v[pl.ds(i * _LANE, _LANE)] = lax.shift_right_logical(r, 1)
        return ()

    lax.fori_loop(0, _BPW // _LANE, _shift, (), unroll=False)
    sems = (s0, s1)
    cp = [None, None]
    cp[0] = pltpu.async_copy(
        fw_hbm.at[idxf_v.at[pl.ds(0, _FB)]], g_v.at[0], sems[0])
    lanes = lax.iota(jnp.int32, _LANE)
    for i in range(_FCH):
        if i + 1 < _FCH:
            cp[(i + 1) % 2] = pltpu.async_copy(
                fw_hbm.at[idxf_v.at[pl.ds((i + 1) * _FB, _FB)]],
                g_v.at[(i + 1) % 2], sems[(i + 1) % 2])
        cp[i % 2].wait()
        gbuf = g_v.at[i % 2]

        def _fext(k, _):
            rg = k // _DIM
            d = k % _DIM
            rows = lanes + rg * _LANE
            r = idx_v[pl.ds(i * _FB + rg * _LANE, _LANE)]
            off = lax.shift_left(lax.bitwise_and(r, 1), 6) + d
            vals = plsc.load_gather(gbuf, [rows, off])
            plsc.store_scatter(ft_v, [lanes * 0 + d, rows], vals)
            return ()

        lax.fori_loop(0, (_FB // _LANE) * _DIM, _fext, (), unroll=False)
        pltpu.sync_copy(ft_v, out_hbm.at[:, pl.ds(base + i * _FB, _FB)])


@functools.partial(
    pl.kernel,
    mesh=_MESH,
    compiler_params=_PARAMS,
    out_type=jax.ShapeDtypeStruct((_K, _N), jnp.float32),
    scratch_types=[
        pltpu.VMEM((_BPW,), jnp.int32),
        pltpu.VMEM((_BPW,), jnp.int32),
        pltpu.VMEM((2, _MB, 128), jnp.float32),
        pltpu.VMEM((_K, _MB), jnp.float32),
        pltpu.SemaphoreType.DMA,
        pltpu.SemaphoreType.DMA,
    ],
)
def _gather_m(ranks_hbm, mm_hbm, out_hbm, idx_v, idxm_v, h_v, mt_v, s0, s1):
    base = _base()
    pltpu.sync_copy(ranks_hbm.at[pl.ds(base, _BPW)], idx_v)

    def _shift(i, _):
        r = idx_v[pl.ds(i * _LANE, _LANE)]
        idxm_v[pl.ds(i * _LANE, _LANE)] = lax.shift_right_logical(r, 5)
        return ()

    lax.fori_loop(0, _BPW // _LANE, _shift, (), unroll=False)
    sems = (s0, s1)
    cp = [None, None]
    cp[0] = pltpu.async_copy(
        mm_hbm.at[idxm_v.at[pl.ds(0, _MB)]], h_v.at[0], sems[0])
    lanes = lax.iota(jnp.int32, _LANE)
    for i in range(_MCH):
        if i + 1 < _MCH:
            cp[(i + 1) % 2] = pltpu.async_copy(
                mm_hbm.at[idxm_v.at[pl.ds((i + 1) * _MB, _MB)]],
                h_v.at[(i + 1) % 2], sems[(i + 1) % 2])
        cp[i % 2].wait()
        hbuf = h_v.at[i % 2]

        def _mext(k, _):
            rg = k // _K
            t = k % _K
            rows = lanes + rg * _LANE
            r = idx_v[pl.ds(i * _MB + rg * _LANE, _LANE)]
            off = lax.shift_left(lax.bitwise_and(r, 31), 2) + t
            vals = plsc.load_gather(hbuf, [rows, off])
            plsc.store_scatter(mt_v, [lanes * 0 + t, rows], vals)
            return ()

        lax.fori_loop(0, (_MB // _LANE) * _K, _mext, (), unroll=False)
        pltpu.sync_copy(mt_v, out_hbm.at[:, pl.ds(base + i * _MB, _MB)])


@jax.jit
def _fire_word(ranks, func_weight, measure_x, measure_m):
    fw2 = func_weight.reshape(_VOCAB // 2, 128)
    mx_t = measure_x.transpose(1, 2, 0).reshape(_K * _DIM, _VOCAB)
    mx2 = mx_t.T
    mm2 = measure_m.reshape(_VOCAB * _K // 128, 128)
    x_rows = _gather_x(ranks, mx2)
    f_t = _gather_f(ranks, fw2)
    m_t = _gather_m(ranks, mm2)
    return (f_t.T, x_rows.reshape(_N, _K, _DIM), m_t.T)


def kernel(ranks, func_weight, measure_x, measure_m):
    return _fire_word(ranks, func_weight, measure_x, measure_m)


# speedup vs baseline: 2.0753x; 1.2278x over previous
import functools

import jax
import jax.numpy as jnp
from jax import lax
from jax.experimental import pallas as pl
from jax.experimental.pallas import tpu as pltpu
from jax.experimental.pallas import tpu_sc as plsc

_VOCAB = 100000
_DIM = 64
_K = 4
_N = 16384

_NC = 2
_NS = 16
_NW = _NC * _NS
_BPW = _N // _NW
_LANE = 16

_XCH = 8
_XB = _BPW // _XCH
_FCH = 4
_FB = _BPW // _FCH
_MCH = 4
_MB = _BPW // _MCH

_MESH = plsc.VectorSubcoreMesh(core_axis_name="c", subcore_axis_name="s")
_PARAMS = pltpu.CompilerParams(use_tc_tiling_on_sc=True,
                               needs_layout_passes=False)


def _base():
    wid = lax.axis_index("s") * _NC + lax.axis_index("c")
    return wid * _BPW


@functools.partial(
    pl.kernel,
    mesh=_MESH,
    compiler_params=_PARAMS,
    out_type=jax.ShapeDtypeStruct((_N, _K * _DIM), jnp.float32),
    scratch_types=[
        pltpu.VMEM((_BPW,), jnp.int32),
        pltpu.VMEM((3, _XB, _K * _DIM), jnp.float32),
        pltpu.SemaphoreType.DMA,
        pltpu.SemaphoreType.DMA,
        pltpu.SemaphoreType.DMA,
        pltpu.SemaphoreType.DMA,
    ],
)
def _gather_x(ranks_hbm, mx_hbm, out_hbm, idx_v, x_v, s0, s1, s2, so):
    base = _base()
    pltpu.sync_copy(ranks_hbm.at[pl.ds(base, _BPW)], idx_v)
    sems = (s0, s1, s2)
    cp = [None] * _XCH
    ocp = [None] * _XCH
    for i in range(2):
        cp[i] = pltpu.async_copy(
            mx_hbm.at[idx_v.at[pl.ds(i * _XB, _XB)]], x_v.at[i], sems[i])
    for i in range(_XCH):
        cp[i].wait()
        if i + 2 < _XCH:
            if i >= 1:
                ocp[i - 1].wait()
            cp[i + 2] = pltpu.async_copy(
                mx_hbm.at[idx_v.at[pl.ds((i + 2) * _XB, _XB)]],
                x_v.at[(i + 2) % 3], sems[(i + 2) % 3])
        ocp[i] = pltpu.async_copy(
            x_v.at[i % 3], out_hbm.at[pl.ds(base + i * _XB, _XB)], so)
    for i in range(max(_XCH - 3, 0), _XCH):
        ocp[i].wait()


@functools.partial(
    pl.kernel,
    mesh=_MESH,
    compiler_params=_PARAMS,
    out_type=(
        jax.ShapeDtypeStruct((_DIM, _N), jnp.float32),
        jax.ShapeDtypeStruct((_K, _N), jnp.float32),
    ),
    scratch_types=[
        pltpu.VMEM((_BPW,), jnp.int32),
        pltpu.VMEM((2, _FB, 128), jnp.float32),
        pltpu.VMEM((_DIM, _FB), jnp.float32),
        pltpu.VMEM((_K, _FB), jnp.float32),
        pltpu.SemaphoreType.DMA,
        pltpu.SemaphoreType.DMA,
    ],
)
def _gather_fm(ranks_hbm, fm_hbm, out_f_hbm, out_m_hbm,
               idx_v, g_v, ft_v, mt_v, s0, s1):
    base = _base()
    pltpu.sync_copy(ranks_hbm.at[pl.ds(base, _BPW)], idx_v)
    sems = (s0, s1)
    cp = [None, None]
    cp[0] = pltpu.async_copy(
        fm_hbm.at[idx_v.at[pl.ds(0, _FB)]], g_v.at[0], sems[0])
    lanes = lax.iota(jnp.int32, _LANE)
    for i in range(_FCH):
        if i + 1 < _FCH:
            cp[(i + 1) % 2] = pltpu.async_copy(
                fm_hbm.at[idx_v.at[pl.ds((i + 1) * _FB, _FB)]],
                g_v.at[(i + 1) % 2], sems[(i + 1) % 2])
        cp[i % 2].wait()
        gbuf = g_v.at[i % 2]

        def _extf(k, _):
            rg = k // _DIM
            d = k % _DIM
            rows = lanes + rg * _LANE
            vals = plsc.load_gather(gbuf, [rows, lanes * 0 + d])
            plsc.store_scatter(ft_v, [lanes * 0 + d, rows], vals)
            return ()

        def _extm(k, _):
            rg = k // _K
            t = k % _K
            rows = lanes + rg * _LANE
            vals = plsc.load_gather(gbuf, [rows, lanes * 0 + (_DIM + t)])
            plsc.store_scatter(mt_v, [lanes * 0 + t, rows], vals)
            return ()

        lax.fori_loop(0, (_FB // _LANE) * _DIM, _extf, (), unroll=False)
        lax.fori_loop(0, (_FB // _LANE) * _K, _extm, (), unroll=False)
        pltpu.sync_copy(ft_v, out_f_hbm.at[:, pl.ds(base + i * _FB, _FB)])
        pltpu.sync_copy(mt_v, out_m_hbm.at[:, pl.ds(base + i * _FB, _FB)])


@jax.jit
def _fire_word(ranks, func_weight, measure_x, measure_m):
    mx_t = measure_x.transpose(1, 2, 0).reshape(_K * _DIM, _VOCAB)
    mx2 = mx_t.T
    fm = jnp.pad(jnp.concatenate([func_weight, measure_m], axis=1),
                 ((0, 0), (0, 128 - _DIM - _K)))
    x_rows = _gather_x(ranks, mx2)
    f_t, m_t = _gather_fm(ranks, fm)
    return (f_t.T, x_rows.reshape(_N, _K, _DIM), m_t.T)


def kernel(ranks, func_weight, measure_x, measure_m):
    return _fire_word(ranks, func_weight, measure_x, measure_m)


# speedup vs baseline: 2.0766x; 1.0006x over previous
import functools

import jax
import jax.numpy as jnp
from jax import lax
from jax.experimental import pallas as pl
from jax.experimental.pallas import tpu as pltpu
from jax.experimental.pallas import tpu_sc as plsc

_VOCAB = 100000
_DIM = 64
_K = 4
_N = 16384

_NC = 2
_NS = 16
_NW = _NC * _NS
_BPW = _N // _NW
_LANE = 16

_XCH = 8
_XB = _BPW // _XCH
_FCH = 4
_FB = _BPW // _FCH
_MCH = 4
_MB = _BPW // _MCH

_MESH = plsc.VectorSubcoreMesh(core_axis_name="c", subcore_axis_name="s")
_PARAMS = pltpu.CompilerParams(use_tc_tiling_on_sc=True,
                               needs_layout_passes=False)


def _base():
    wid = lax.axis_index("s") * _NC + lax.axis_index("c")
    return wid * _BPW


@functools.partial(
    pl.kernel,
    mesh=_MESH,
    compiler_params=_PARAMS,
    out_type=jax.ShapeDtypeStruct((_N, _K * _DIM), jnp.float32),
    scratch_types=[
        pltpu.VMEM((_BPW,), jnp.int32),
        pltpu.VMEM((3, _XB, _K * _DIM), jnp.float32),
        pltpu.SemaphoreType.DMA,
        pltpu.SemaphoreType.DMA,
        pltpu.SemaphoreType.DMA,
        pltpu.SemaphoreType.DMA,
    ],
)
def _gather_x(ranks_hbm, mx_hbm, out_hbm, idx_v, x_v, s0, s1, s2, so):
    base = _base()
    pltpu.sync_copy(ranks_hbm.at[pl.ds(base, _BPW)], idx_v)
    sems = (s0, s1, s2)
    cp = [None] * _XCH
    ocp = [None] * _XCH
    for i in range(2):
        cp[i] = pltpu.async_copy(
            mx_hbm.at[idx_v.at[pl.ds(i * _XB, _XB)]], x_v.at[i], sems[i])
    for i in range(_XCH):
        cp[i].wait()
        if i + 2 < _XCH:
            if i >= 1:
                ocp[i - 1].wait()
            cp[i + 2] = pltpu.async_copy(
                mx_hbm.at[idx_v.at[pl.ds((i + 2) * _XB, _XB)]],
                x_v.at[(i + 2) % 3], sems[(i + 2) % 3])
        ocp[i] = pltpu.async_copy(
            x_v.at[i % 3], out_hbm.at[pl.ds(base + i * _XB, _XB)], so)
    for i in range(max(_XCH - 3, 0), _XCH):
        ocp[i].wait()


@functools.partial(
    pl.kernel,
    mesh=_MESH,
    compiler_params=_PARAMS,
    out_type=(
        jax.ShapeDtypeStruct((_DIM, _N), jnp.float32),
        jax.ShapeDtypeStruct((_K, _N), jnp.float32),
    ),
    scratch_types=[
        pltpu.VMEM((_BPW,), jnp.int32),
        pltpu.VMEM((2, _FB, 128), jnp.float32),
        pltpu.VMEM((_DIM, _FB), jnp.float32),
        pltpu.VMEM((_K, _FB), jnp.float32),
        pltpu.SemaphoreType.DMA,
        pltpu.SemaphoreType.DMA,
    ],
)
def _gather_fm(ranks_hbm, fm_hbm, out_f_hbm, out_m_hbm,
               idx_v, g_v, ft_v, mt_v, s0, s1):
    base = _base()
    pltpu.sync_copy(ranks_hbm.at[pl.ds(base, _BPW)], idx_v)
    sems = (s0, s1)
    cp = [None, None]
    cp[0] = pltpu.async_copy(
        fm_hbm.at[idx_v.at[pl.ds(0, _FB)]], g_v.at[0], sems[0])
    lanes = lax.iota(jnp.int32, _LANE)
    for i in range(_FCH):
        if i + 1 < _FCH:
            cp[(i + 1) % 2] = pltpu.async_copy(
                fm_hbm.at[idx_v.at[pl.ds((i + 1) * _FB, _FB)]],
                g_v.at[(i + 1) % 2], sems[(i + 1) % 2])
        cp[i % 2].wait()
        gbuf = g_v.at[i % 2]

        def _extf(k, _):
            rg = k // _DIM
            d = k % _DIM
            rows = lanes + rg * _LANE
            vals = plsc.load_gather(gbuf, [rows, lanes * 0 + d])
            plsc.store_scatter(ft_v, [lanes * 0 + d, rows], vals)
            return ()

        def _extm(k, _):
            rg = k // _K
            t = k % _K
            rows = lanes + rg * _LANE
            vals = plsc.load_gather(gbuf, [rows, lanes * 0 + (_DIM + t)])
            plsc.store_scatter(mt_v, [lanes * 0 + t, rows], vals)
            return ()

        lax.fori_loop(0, (_FB // _LANE) * _DIM, _extf, (), unroll=False)
        lax.fori_loop(0, (_FB // _LANE) * _K, _extm, (), unroll=False)
        pltpu.sync_copy(ft_v, out_f_hbm.at[:, pl.ds(base + i * _FB, _FB)])
        pltpu.sync_copy(mt_v, out_m_hbm.at[:, pl.ds(base + i * _FB, _FB)])


@jax.jit
def _fire_word(ranks, func_weight, measure_x, measure_m):
    mx_t = measure_x.transpose(1, 2, 0).reshape(_K * _DIM, _VOCAB)
    mx2 = mx_t.T
    fm = jnp.concatenate(
        [func_weight, measure_m,
         jnp.zeros((_VOCAB, 128 - _DIM - _K), jnp.float32)], axis=1)
    x_rows = _gather_x(ranks, mx2)
    f_t, m_t = _gather_fm(ranks, fm)
    return (f_t.T, x_rows.reshape(_N, _K, _DIM), m_t.T)


def kernel(ranks, func_weight, measure_x, measure_m):
    return _fire_word(ranks, func_weight, measure_x, measure_m)
